# baseline (speedup 1.0000x reference)
import jax
import jax.numpy as jnp
from jax.experimental import pallas as pl
from jax.experimental.pallas import tpu as pltpu

N_CLS = 1000
N_CTX = 4
DIM = 512
CTX_LEN = 77
SUFFIX_LEN = CTX_LEN - 1 - N_CTX
PAD_LEN = 80

G = 50


def _concat_kernel(pref_ref, ctx_ref, suf_ref, out_ref):
    out_ref[:, 0:1, :] = pref_ref[...]
    ctx = ctx_ref[...]
    out_ref[:, 1:1 + N_CTX, :] = jnp.broadcast_to(ctx[None, :, :], (G, N_CTX, DIM))
    out_ref[:, 1 + N_CTX:1 + N_CTX + SUFFIX_LEN, :] = suf_ref[...]
    out_ref[:, CTX_LEN:, :] = suf_ref[:, 0:PAD_LEN - CTX_LEN, :]


def kernel(prefixs, ctx, suffixs):
    grid = (N_CLS // G,)
    return pl.pallas_call(
        _concat_kernel,
        grid=grid,
        in_specs=[
            pl.BlockSpec((G, 1, DIM), lambda i: (i, 0, 0)),
            pl.BlockSpec((N_CTX, DIM), lambda i: (0, 0)),
            pl.BlockSpec((G, SUFFIX_LEN, DIM), lambda i: (i, 0, 0)),
        ],
        out_specs=pl.BlockSpec((G, PAD_LEN, DIM), lambda i: (i, 0, 0)),
        out_shape=jax.ShapeDtypeStruct((N_CLS, CTX_LEN, DIM), jnp.float32),
        compiler_params=pltpu.CompilerParams(
            dimension_semantics=("arbitrary",),
        ),
    )(prefixs, ctx, suffixs)


# speedup vs baseline: 1.0132x; 1.0132x over previous
import jax
import jax.numpy as jnp
from jax.experimental import pallas as pl
from jax.experimental.pallas import tpu as pltpu

N_CLS = 1000
N_CTX = 4
DIM = 512
CTX_LEN = 77
SUFFIX_LEN = CTX_LEN - 1 - N_CTX
SPLIT = 72

G = 50


def _concat_kernel(pref_ref, ctx_ref, suf_ref, out_hbm, scratch, semA, semB):
    i = pl.program_id(0)
    n = pl.num_programs(0)
    s = jax.lax.rem(i, 2)

    def copies(step, slot):
        base = step * G
        cA = pltpu.make_async_copy(
            scratch.at[slot, :, 0:SPLIT, :],
            out_hbm.at[pl.ds(base, G), 0:SPLIT, :],
            semA.at[slot])
        cB = pltpu.make_async_copy(
            scratch.at[slot, :, SPLIT:CTX_LEN, :],
            out_hbm.at[pl.ds(base, G), SPLIT:CTX_LEN, :],
            semB.at[slot])
        return cA, cB

    @pl.when(i >= 2)
    def _wait_prev():
        cA, cB = copies(i - 2, s)
        cA.wait()
        cB.wait()

    scratch[s, :, 0:1, :] = pref_ref[...]
    ctx = ctx_ref[...]
    scratch[s, :, 1:1 + N_CTX, :] = jnp.broadcast_to(ctx[None, :, :],
                                                     (G, N_CTX, DIM))
    scratch[s, :, 1 + N_CTX:, :] = suf_ref[...]

    cA, cB = copies(i, s)
    cA.start()
    cB.start()

    @pl.when(i == n - 1)
    def _drain():
        dA, dB = copies(i, s)
        dA.wait()
        dB.wait()
        eA, eB = copies(i - 1, 1 - s)
        eA.wait()
        eB.wait()


def kernel(prefixs, ctx, suffixs):
    grid = (N_CLS // G,)
    return pl.pallas_call(
        _concat_kernel,
        grid=grid,
        in_specs=[
            pl.BlockSpec((G, 1, DIM), lambda i: (i, 0, 0)),
            pl.BlockSpec((N_CTX, DIM), lambda i: (0, 0)),
            pl.BlockSpec((G, SUFFIX_LEN, DIM), lambda i: (i, 0, 0)),
        ],
        out_specs=pl.BlockSpec(memory_space=pl.ANY),
        out_shape=jax.ShapeDtypeStruct((N_CLS, CTX_LEN, DIM), jnp.float32),
        scratch_shapes=[
            pltpu.VMEM((2, G, CTX_LEN, DIM), jnp.float32),
            pltpu.SemaphoreType.DMA((2,)),
            pltpu.SemaphoreType.DMA((2,)),
        ],
        compiler_params=pltpu.CompilerParams(
            dimension_semantics=("arbitrary",),
        ),
    )(prefixs, ctx, suffixs)


# speedup vs baseline: 1.0147x; 1.0015x over previous
import jax
import jax.numpy as jnp
from jax.experimental import pallas as pl
from jax.experimental.pallas import tpu as pltpu

N_CLS = 1000
N_CTX = 4
DIM = 512
CTX_LEN = 77
SUFFIX_LEN = CTX_LEN - 1 - N_CTX
PAD_LEN = 80

G = 50


def _concat_kernel(pref_ref, ctx_ref, suf_ref, out_hbm, scratch, sem):
    i = pl.program_id(0)
    n = pl.num_programs(0)
    s = jax.lax.rem(i, 2)

    def copy(step, slot):
        return pltpu.make_async_copy(
            scratch.at[slot],
            out_hbm.at[pl.ds(step * G, G), pl.ds(0, PAD_LEN), :],
            sem.at[slot])

    @pl.when(i >= 2)
    def _wait_prev():
        copy(i - 2, s).wait()

    scratch[s, :, 0:1, :] = pref_ref[...]
    ctx = ctx_ref[...]
    scratch[s, :, 1:1 + N_CTX, :] = jnp.broadcast_to(ctx[None, :, :],
                                                     (G, N_CTX, DIM))
    scratch[s, :, 1 + N_CTX:CTX_LEN, :] = suf_ref[...]
    scratch[s, :, CTX_LEN:, :] = suf_ref[:, 0:PAD_LEN - CTX_LEN, :]

    copy(i, s).start()

    @pl.when(i == n - 1)
    def _drain():
        copy(i, s).wait()
        copy(i - 1, 1 - s).wait()


def kernel(prefixs, ctx, suffixs):
    grid = (N_CLS // G,)
    return pl.pallas_call(
        _concat_kernel,
        grid=grid,
        in_specs=[
            pl.BlockSpec((G, 1, DIM), lambda i: (i, 0, 0)),
            pl.BlockSpec((N_CTX, DIM), lambda i: (0, 0)),
            pl.BlockSpec((G, SUFFIX_LEN, DIM), lambda i: (i, 0, 0)),
        ],
        out_specs=pl.BlockSpec(memory_space=pl.ANY),
        out_shape=jax.ShapeDtypeStruct((N_CLS, CTX_LEN, DIM), jnp.float32),
        scratch_shapes=[
            pltpu.VMEM((2, G, PAD_LEN, DIM), jnp.float32),
            pltpu.SemaphoreType.DMA((2,)),
        ],
        compiler_params=pltpu.CompilerParams(
            dimension_semantics=("arbitrary",),
        ),
    )(prefixs, ctx, suffixs)


# speedup vs baseline: 2.0290x; 1.9996x over previous
import jax
import jax.numpy as jnp
from jax.experimental import pallas as pl
from jax.experimental.pallas import tpu as pltpu

N_CLS = 1000
N_CTX = 4
DIM = 512
CTX_LEN = 77
SUFFIX_LEN = CTX_LEN - 1 - N_CTX

G = 40


def _concat_kernel(pref_ref, ctx_ref, suf_ref, out_ref):
    out_ref[0:1, :, :] = jnp.transpose(pref_ref[...], (1, 0, 2))
    ctx = ctx_ref[...]
    out_ref[1:1 + N_CTX, :, :] = jnp.broadcast_to(ctx[:, None, :],
                                                  (N_CTX, G, DIM))
    out_ref[1 + N_CTX:, :, :] = jnp.transpose(suf_ref[...], (1, 0, 2))


def kernel(prefixs, ctx, suffixs):
    grid = (N_CLS // G,)
    out_t = pl.pallas_call(
        _concat_kernel,
        grid=grid,
        in_specs=[
            pl.BlockSpec((G, 1, DIM), lambda i: (i, 0, 0)),
            pl.BlockSpec((N_CTX, DIM), lambda i: (0, 0)),
            pl.BlockSpec((G, SUFFIX_LEN, DIM), lambda i: (i, 0, 0)),
        ],
        out_specs=pl.BlockSpec((CTX_LEN, G, DIM), lambda i: (0, i, 0)),
        out_shape=jax.ShapeDtypeStruct((CTX_LEN, N_CLS, DIM), jnp.float32),
        compiler_params=pltpu.CompilerParams(
            dimension_semantics=("arbitrary",),
        ),
    )(prefixs, ctx, suffixs)
    return jnp.transpose(out_t, (1, 0, 2))
